# baseline (speedup 1.0000x reference)
import functools

import jax
import jax.numpy as jnp
from jax import lax
from jax.experimental import pallas as pl
from jax.experimental.pallas import tpu as pltpu
from jax.experimental.pallas import tpu_sc as plsc

_NW = 32
_NH = 32

_TOTAL = 16 * 512 * 512
_NWORK = 32
_PER_W = _TOTAL // _NWORK
_CHUNK = 16384
_NCHUNK = _PER_W // _CHUNK
_LANES = 16


def _sc_call(rel_flat, init_flat):
    mesh = plsc.VectorSubcoreMesh(core_axis_name="c", subcore_axis_name="s")

    @functools.partial(
        pl.kernel,
        mesh=mesh,
        out_type=jax.ShapeDtypeStruct((_TOTAL,), jnp.int32),
        scratch_types=[
            pltpu.VMEM((_CHUNK,), jnp.int32),
            pltpu.VMEM((_CHUNK,), jnp.int32),
            pltpu.VMEM((_CHUNK,), jnp.int32),
        ],
    )
    def k(rel_hbm, init_hbm, out_hbm, rel_v, init_v, out_v):
        cid = lax.axis_index("c")
        sid = lax.axis_index("s")
        wid = sid * 2 + cid
        base = wid * _PER_W

        def chunk_body(ci, carry):
            off = base + ci * _CHUNK
            pltpu.sync_copy(rel_hbm.at[pl.ds(off, _CHUNK)], rel_v)
            pltpu.sync_copy(init_hbm.at[pl.ds(off, _CHUNK)], init_v)

            c3 = jnp.full((_LANES,), 3, jnp.int32)
            c5 = jnp.full((_LANES,), 5, jnp.int32)
            c11 = jnp.full((_LANES,), 11, jnp.int32)
            c1 = jnp.full((_LANES,), 1, jnp.int32)
            c0 = jnp.full((_LANES,), 0, jnp.int32)
            cNW = jnp.full((_LANES,), _NW, jnp.int32)
            cM = jnp.full((_LANES,), _NW - 1, jnp.int32)

            def vec_body(vi, c2):
                r = rel_v[pl.ds(vi * _LANES, _LANES)]
                i = init_v[pl.ds(vi * _LANES, _LANES)]
                dr1 = lax.shift_right_logical(lax.mul(r, c11), c5)
                dc1 = lax.sub(r, lax.mul(dr1, c3))
                ir = lax.shift_right_logical(i, c5)
                ic = lax.bitwise_and(i, cM)
                ar = lax.min(lax.max(lax.sub(lax.add(ir, dr1), c1), c0), cM)
                ac = lax.min(lax.max(lax.sub(lax.add(ic, dc1), c1), c0), cM)
                out_v[pl.ds(vi * _LANES, _LANES)] = lax.add(
                    lax.mul(ar, cNW), ac)
                return c2

            lax.fori_loop(0, _CHUNK // _LANES, vec_body, 0)
            pltpu.sync_copy(out_v, out_hbm.at[pl.ds(off, _CHUNK)])
            return carry

        lax.fori_loop(0, _NCHUNK, chunk_body, 0)

    return k(rel_flat, init_flat)


def kernel(rel_idx_map, init_idx_map):
    shape = rel_idx_map.shape
    rel_flat = rel_idx_map.reshape(_TOTAL).astype(jnp.int32)
    init_flat = init_idx_map.reshape(_TOTAL).astype(jnp.int32)
    out = _sc_call(rel_flat, init_flat)
    return out.reshape(shape).astype(rel_idx_map.dtype)


# speedup vs baseline: 1.5720x; 1.5720x over previous
import functools

import jax
import jax.numpy as jnp
import numpy as np
from jax import lax
from jax.experimental import pallas as pl
from jax.experimental.pallas import tpu as pltpu
from jax.experimental.pallas import tpu_sc as plsc

_NW = 32
_NH = 32

_B = 16
_H = 512
_W = 512
_NWORK = 32
_ROWS_PER_W = (_B * _H) // _NWORK
_CHUNK_ROWS = 32
_NCHUNK = _ROWS_PER_W // _CHUNK_ROWS
_LANES = 16
_COL_CHUNKS = _W // _LANES


def _build_lut() -> np.ndarray:
    init = np.arange(_NW * _NH, dtype=np.int64)[:, None]
    rel = np.arange(9, dtype=np.int64)[None, :]
    ir = init // _NW
    ic = init % _NW
    dr = rel // 3 - 1
    dc = rel % 3 - 1
    ar = np.clip(ir + dr, 0, _NH - 1)
    ac = np.clip(ic + dc, 0, _NW - 1)
    return (ar * _NW + ac).astype(np.int32).reshape(-1)


_LUT = _build_lut()


def _sc_call(rel4d, init4d, lut):
    mesh = plsc.VectorSubcoreMesh(core_axis_name="c", subcore_axis_name="s")

    @functools.partial(
        pl.kernel,
        mesh=mesh,
        compiler_params=pltpu.CompilerParams(needs_layout_passes=False),
        out_type=jax.ShapeDtypeStruct((_B, 1, _H, _W), jnp.int32),
        scratch_types=[
            pltpu.VMEM((9216,), jnp.int32),
            pltpu.VMEM((_CHUNK_ROWS, _W), jnp.int32),
            pltpu.VMEM((_CHUNK_ROWS, _W), jnp.int32),
            pltpu.VMEM((_CHUNK_ROWS, _W), jnp.int32),
        ],
    )
    def k(rel_hbm, init_hbm, lut_hbm, out_hbm, lut_v, rel_v, init_v, out_v):
        cid = lax.axis_index("c")
        sid = lax.axis_index("s")
        wid = sid * 2 + cid
        pltpu.sync_copy(lut_hbm, lut_v)

        b = wid // 2
        r0 = (wid % 2) * _ROWS_PER_W

        c9 = jnp.full((_LANES,), 9, jnp.int32)

        def chunk_body(ci, carry):
            rr = r0 + ci * _CHUNK_ROWS
            pltpu.sync_copy(rel_hbm.at[b, 0, pl.ds(rr, _CHUNK_ROWS), :], rel_v)
            pltpu.sync_copy(init_hbm.at[b, 0, pl.ds(rr, _CHUNK_ROWS), :], init_v)

            def row_body(row, c2):
                def col_body(cc, c3):
                    col = cc * _LANES
                    r = rel_v[row, pl.ds(col, _LANES)]
                    i = init_v[row, pl.ds(col, _LANES)]
                    idx = lax.add(lax.mul(i, c9), r)
                    out_v[row, pl.ds(col, _LANES)] = plsc.load_gather(
                        lut_v, [idx])
                    return c3

                return lax.fori_loop(0, _COL_CHUNKS, col_body, c2)

            lax.fori_loop(0, _CHUNK_ROWS, row_body, 0)
            pltpu.sync_copy(out_v, out_hbm.at[b, 0, pl.ds(rr, _CHUNK_ROWS), :])
            return carry

        lax.fori_loop(0, _NCHUNK, chunk_body, 0)

    return k(rel4d, init4d, lut)


def kernel(rel_idx_map, init_idx_map):
    lut = jnp.asarray(_LUT)
    out = _sc_call(rel_idx_map.astype(jnp.int32),
                   init_idx_map.astype(jnp.int32), lut)
    return out.astype(rel_idx_map.dtype)


# speedup vs baseline: 2.9867x; 1.9000x over previous
import functools

import jax
import jax.numpy as jnp
import numpy as np
from jax import lax
from jax.experimental import pallas as pl
from jax.experimental.pallas import tpu as pltpu
from jax.experimental.pallas import tpu_sc as plsc

_NW = 32
_NH = 32

_B = 16
_H = 512
_W = 512
_NWORK = 32
_ROWS_PER_W = (_B * _H) // _NWORK
_CHUNK_ROWS = 32
_NCHUNK = _ROWS_PER_W // _CHUNK_ROWS
_LANES = 16
_COL_CHUNKS = _W // _LANES


def _build_lut() -> np.ndarray:
    init = np.arange(_NW * _NH, dtype=np.int64)[:, None]
    rel = np.arange(9, dtype=np.int64)[None, :]
    ir = init // _NW
    ic = init % _NW
    dr = rel // 3 - 1
    dc = rel % 3 - 1
    ar = np.clip(ir + dr, 0, _NH - 1)
    ac = np.clip(ic + dc, 0, _NW - 1)
    return (ar * _NW + ac).astype(np.int32).reshape(-1)


_LUT = _build_lut()


def _sc_call(rel4d, init4d, lut):
    mesh = plsc.VectorSubcoreMesh(core_axis_name="c", subcore_axis_name="s")

    @functools.partial(
        pl.kernel,
        mesh=mesh,
        compiler_params=pltpu.CompilerParams(needs_layout_passes=False),
        out_type=jax.ShapeDtypeStruct((_B, 1, _H, _W), jnp.int32),
        scratch_types=[
            pltpu.VMEM((9216,), jnp.int32),
            [pltpu.VMEM((_CHUNK_ROWS, _W), jnp.int32)] * 2,
            [pltpu.VMEM((_CHUNK_ROWS, _W), jnp.int32)] * 2,
            [pltpu.VMEM((_CHUNK_ROWS, _W), jnp.int32)] * 2,
            [pltpu.SemaphoreType.DMA] * 6,
        ],
    )
    def k(rel_hbm, init_hbm, lut_hbm, out_hbm, lut_v, rel_b, init_b, out_b,
          sems):
        cid = lax.axis_index("c")
        sid = lax.axis_index("s")
        wid = sid * 2 + cid
        pltpu.sync_copy(lut_hbm, lut_v)

        b = wid // 2
        r0 = (wid % 2) * _ROWS_PER_W

        c9 = jnp.full((_LANES,), 9, jnp.int32)
        sh9 = jnp.int32(9)
        m511 = jnp.int32(_W - 1)

        def hslice(ref, g):
            return ref.at[b, 0, pl.ds(r0 + g * _CHUNK_ROWS, _CHUNK_ROWS), :]

        def start_in(g):
            bb = g % 2
            return (
                pltpu.async_copy(hslice(rel_hbm, g), rel_b[bb], sems[bb]),
                pltpu.async_copy(hslice(init_hbm, g), init_b[bb], sems[2 + bb]),
            )

        in_copies = {}
        out_copies = {}
        in_copies[0] = start_in(0)
        for g in range(_NCHUNK):
            bb = g % 2
            if g + 1 < _NCHUNK:
                in_copies[g + 1] = start_in(g + 1)
            in_copies[g][0].wait()
            in_copies[g][1].wait()
            if g >= 2:
                out_copies[g - 2].wait()

            rel_v = rel_b[bb]
            init_v = init_b[bb]
            out_v = out_b[bb]

            @plsc.parallel_loop(0, _CHUNK_ROWS * _W, step=_LANES, unroll=8)
            def body(v):
                row = lax.shift_right_logical(v, sh9)
                col = lax.bitwise_and(v, m511)
                r = rel_v[row, pl.ds(col, _LANES)]
                i = init_v[row, pl.ds(col, _LANES)]
                idx = lax.add(lax.mul(i, c9), r)
                out_v[row, pl.ds(col, _LANES)] = plsc.load_gather(
                    lut_v, [idx])

            out_copies[g] = pltpu.async_copy(
                out_v, hslice(out_hbm, g), sems[4 + bb])

        out_copies[_NCHUNK - 2].wait()
        out_copies[_NCHUNK - 1].wait()

    return k(rel4d, init4d, lut)


def kernel(rel_idx_map, init_idx_map):
    lut = jnp.asarray(_LUT)
    out = _sc_call(rel_idx_map.astype(jnp.int32),
                   init_idx_map.astype(jnp.int32), lut)
    return out.astype(rel_idx_map.dtype)
